# Initial kernel scaffold; baseline (speedup 1.0000x reference)
import jax
import jax.numpy as jnp
from jax.experimental import pallas as pl


def kernel(x, edge_index, edge_attr, graph_features, A1, b1, Wr1, br1, A2, b2, Wr2, br2, Wh, bh):
    raise NotImplementedError("write your pallas kernel here")


# speedup vs baseline: 3.8989x; 3.8989x over previous
import functools

import jax
import jax.numpy as jnp
from jax import lax
from jax.experimental import pallas as pl
from jax.experimental.pallas import tpu as pltpu
from jax.experimental.pallas import tpu_sc as plsc

N = 10000
E = 160000
DIN = 128
H = 16
DE = 16
GF = 8

PW = DE * H + H
C = 128
NP = 10240

_info = plsc.get_sparse_core_info()
NC, NS = _info.num_cores, _info.num_subcores
NW = NC * NS


@functools.partial(
    pl.kernel,
    out_type=jax.ShapeDtypeStruct((NC, NP, 2 * H), jnp.float32),
    mesh=plsc.VectorSubcoreMesh(core_axis_name="c", subcore_axis_name="s"),
    scratch_types=[
        pltpu.VMEM((C,), jnp.int32),
        pltpu.VMEM((C,), jnp.int32),
        pltpu.VMEM((C, PW), jnp.float32),
        pltpu.VMEM((C, DE), jnp.float32),
        pltpu.VMEM((C, 2 * H), jnp.float32),
        pltpu.VMEM_SHARED((NP, 2 * H), jnp.float32),
        pltpu.SemaphoreType.DMA,
    ],
    compiler_params=pltpu.CompilerParams(use_tc_tiling_on_sc=False),
)
def _edge_pass(p_hbm, src_hbm, dst_hbm, ea_hbm, zeros_hbm, out_hbm,
               idx_v, dst_v, rows_v, ea_v, msg_v, acc_sh, sem):
    c = lax.axis_index("c")
    s = lax.axis_index("s")
    wid = s * NC + c

    rows_per = NP // NS
    stripe = pl.multiple_of(s * rows_per, 8)
    pltpu.sync_copy(zeros_hbm.at[pl.ds(stripe, rows_per)],
                    acc_sh.at[pl.ds(stripe, rows_per)])

    cvec = jnp.where(lax.iota(jnp.int32, H) == 0,
                     jnp.float32(1.0), jnp.float32(0.0))

    def init_body(e, carry):
        msg_v[e, pl.ds(H, H)] = cvec
        return carry

    lax.fori_loop(0, C, init_body, 0)
    plsc.subcore_barrier()

    nchunks = E // C
    niter = (nchunks + NW - 1) // NW

    def chunk_body(i, carry):
        cid = wid + i * NW

        @pl.when(cid < nchunks)
        def _():
            base = pl.multiple_of(cid * C, C)
            pltpu.sync_copy(src_hbm.at[pl.ds(base, C)], idx_v)
            pltpu.sync_copy(dst_hbm.at[pl.ds(base, C)], dst_v)
            pltpu.sync_copy(ea_hbm.at[pl.ds(base, C)], ea_v)
            pltpu.async_copy(p_hbm.at[idx_v], rows_v, sem).wait()

            def edge_body(e, carry2):
                acc = rows_v[e, pl.ds(DE * H, H)]
                eav = ea_v[e, pl.ds(0, DE)]
                for k in range(DE):
                    acc = acc + eav[k] * rows_v[e, pl.ds(k * H, H)]
                msg_v[e, pl.ds(0, H)] = acc
                return carry2

            lax.fori_loop(0, C, edge_body, 0)
            pltpu.sync_copy(msg_v, acc_sh.at[dst_v], add=True)

        return carry

    lax.fori_loop(0, niter, chunk_body, 0)
    plsc.subcore_barrier()

    pltpu.sync_copy(acc_sh.at[pl.ds(stripe, rows_per)],
                    out_hbm.at[c, pl.ds(stripe, rows_per)])


_BN = 2000


def _dense1_body(x_ref, w_ref, b_ref, p_ref, r_ref):
    acc = jnp.dot(x_ref[...], w_ref[...], preferred_element_type=jnp.float32)
    p_ref[...] = acc[:, :PW]
    r_ref[...] = acc[:, PW:] + b_ref[...]


def _mid_body(pp_ref, r1_ref, w_ref, b_ref, p2_ref, r2_ref):
    pa = pp_ref[0]
    pb = pp_ref[1]
    ssum = pa[:, :H] + pb[:, :H]
    cnt = pa[:, H:H + 1] + pb[:, H:H + 1]
    h = jnp.maximum(ssum / jnp.maximum(cnt, 1.0) + r1_ref[...], 0.0)
    acc = jnp.dot(h, w_ref[...], preferred_element_type=jnp.float32)
    p2_ref[...] = acc[:, :PW]
    r2_ref[...] = acc[:, PW:] + b_ref[...]


def _head_body(pp_ref, r2_ref, g_ref, wh_ref, wg_ref, bh_ref, o_ref):
    pa = pp_ref[0]
    pb = pp_ref[1]
    ssum = pa[:, :H] + pb[:, :H]
    cnt = pa[:, H:H + 1] + pb[:, H:H + 1]
    h = jnp.maximum(ssum / jnp.maximum(cnt, 1.0) + r2_ref[...], 0.0)
    o_ref[...] = (jnp.dot(h, wh_ref[...], preferred_element_type=jnp.float32)
                  + jnp.dot(g_ref[...], wg_ref[...],
                            preferred_element_type=jnp.float32)
                  + bh_ref[...])


def _row_spec(width):
    return pl.BlockSpec((_BN, width), lambda i: (i, 0))


def _full_spec(shape):
    return pl.BlockSpec(shape, lambda i: tuple(0 for _ in shape))


_GRID = (N // _BN,)

_dense1 = pl.pallas_call(
    _dense1_body,
    grid=_GRID,
    in_specs=[_row_spec(DIN), _full_spec((DIN, PW + H)), _full_spec((1, H))],
    out_specs=[_row_spec(PW), _row_spec(H)],
    out_shape=[jax.ShapeDtypeStruct((N, PW), jnp.float32),
               jax.ShapeDtypeStruct((N, H), jnp.float32)],
)

_mid = pl.pallas_call(
    _mid_body,
    grid=_GRID,
    in_specs=[pl.BlockSpec((NC, _BN, 2 * H), lambda i: (0, i, 0)),
              _row_spec(H), _full_spec((H, PW + H)), _full_spec((1, H))],
    out_specs=[_row_spec(PW), _row_spec(H)],
    out_shape=[jax.ShapeDtypeStruct((N, PW), jnp.float32),
               jax.ShapeDtypeStruct((N, H), jnp.float32)],
)

_head = pl.pallas_call(
    _head_body,
    grid=_GRID,
    in_specs=[pl.BlockSpec((NC, _BN, 2 * H), lambda i: (0, i, 0)),
              _row_spec(H), _row_spec(GF), _full_spec((H, 1)),
              _full_spec((GF, 1)), _full_spec((1, 1))],
    out_specs=_row_spec(1),
    out_shape=jax.ShapeDtypeStruct((N, 1), jnp.float32),
)


def kernel(x, edge_index, edge_attr, graph_features,
           A1, b1, Wr1, br1, A2, b2, Wr2, br2, Wh, bh):
    W1 = jnp.concatenate([
        A1.reshape(DE, DIN, H).transpose(1, 0, 2).reshape(DIN, DE * H),
        b1.reshape(DIN, H), Wr1], axis=1)
    W2 = jnp.concatenate([
        A2.reshape(DE, H, H).transpose(1, 0, 2).reshape(H, DE * H),
        b2.reshape(H, H), Wr2], axis=1)

    src = edge_index[0]
    dst = edge_index[1]
    zeros = jnp.zeros((NP, 2 * H), jnp.float32)

    p1, root1 = _dense1(x, W1, br1.reshape(1, H))
    part1 = _edge_pass(p1, src, dst, edge_attr, zeros)
    p2, root2 = _mid(part1, root1, W2, br2.reshape(1, H))
    part2 = _edge_pass(p2, src, dst, edge_attr, zeros)
    return _head(part2, root2, graph_features.T,
                 Wh[:H], Wh[H:], bh.reshape(1, 1))


# speedup vs baseline: 4.9973x; 1.2817x over previous
import functools

import jax
import jax.numpy as jnp
from jax import lax
from jax.experimental import pallas as pl
from jax.experimental.pallas import tpu as pltpu
from jax.experimental.pallas import tpu_sc as plsc

N = 10000
E = 160000
DIN = 128
H = 16
DE = 16
GF = 8

PW = DE * H + H
C = 128
NP = 10240

_info = plsc.get_sparse_core_info()
NC, NS = _info.num_cores, _info.num_subcores
NW = NC * NS


@functools.partial(
    pl.kernel,
    out_type=jax.ShapeDtypeStruct((NC, NP, 2 * H), jnp.float32),
    mesh=plsc.VectorSubcoreMesh(core_axis_name="c", subcore_axis_name="s"),
    scratch_types=[
        [pltpu.VMEM((C,), jnp.int32)] * 2,
        [pltpu.VMEM((C,), jnp.int32)] * 2,
        [pltpu.VMEM((C, PW), jnp.float32)] * 2,
        [pltpu.VMEM((C, DE), jnp.float32)] * 2,
        [pltpu.VMEM((C, 2 * H), jnp.float32)] * 2,
        pltpu.VMEM_SHARED((NP, 2 * H), jnp.float32),
        [pltpu.SemaphoreType.DMA] * 2,
    ],
    compiler_params=pltpu.CompilerParams(use_tc_tiling_on_sc=False),
)
def _edge_pass(p_hbm, src_hbm, dst_hbm, ea_hbm, zeros_hbm, out_hbm,
               idx_v, dst_v, rows_v, ea_v, msg_v, acc_sh, sem):
    c = lax.axis_index("c")
    s = lax.axis_index("s")
    wid = s * NC + c

    rows_per = NP // NS
    stripe = pl.multiple_of(s * rows_per, 8)
    pltpu.sync_copy(zeros_hbm.at[pl.ds(stripe, rows_per)],
                    acc_sh.at[pl.ds(stripe, rows_per)])

    cvec = jnp.where(lax.iota(jnp.int32, H) == 0,
                     jnp.float32(1.0), jnp.float32(0.0))

    def init_body(e, carry):
        msg_v[0][e, pl.ds(H, H)] = cvec
        msg_v[1][e, pl.ds(H, H)] = cvec
        return carry

    lax.fori_loop(0, C, init_body, 0)
    plsc.subcore_barrier()

    nchunks = E // C
    niter = (nchunks + NW - 1) // NW

    def start(j, b):
        cid = wid + j * NW

        @pl.when(cid < nchunks)
        def _():
            base = pl.multiple_of(cid * C, C)
            pltpu.sync_copy(src_hbm.at[pl.ds(base, C)], idx_v[b])
            pltpu.sync_copy(dst_hbm.at[pl.ds(base, C)], dst_v[b])
            pltpu.sync_copy(ea_hbm.at[pl.ds(base, C)], ea_v[b])
            pltpu.async_copy(p_hbm.at[idx_v[b]], rows_v[b], sem[b])

    def process(j, b):
        cid = wid + j * NW

        @pl.when(cid < nchunks)
        def _():
            pltpu.make_async_copy(p_hbm.at[idx_v[b]], rows_v[b],
                                  sem[b]).wait()

            def edge_body(e, carry2):
                acc = rows_v[b][e, pl.ds(DE * H, H)]
                eav = ea_v[b][e, pl.ds(0, DE)]
                for k in range(DE):
                    acc = acc + eav[k] * rows_v[b][e, pl.ds(k * H, H)]
                msg_v[b][e, pl.ds(0, H)] = acc
                return carry2

            lax.fori_loop(0, C, edge_body, 0)
            pltpu.sync_copy(msg_v[b], acc_sh.at[dst_v[b]], add=True)

    start(0, 0)
    start(1, 1)

    def chunk_body(t, carry):
        j = 2 * t
        process(j, 0)
        start(j + 2, 0)
        process(j + 1, 1)
        start(j + 3, 1)
        return carry

    lax.fori_loop(0, niter // 2, chunk_body, 0)
    plsc.subcore_barrier()

    pltpu.sync_copy(acc_sh.at[pl.ds(stripe, rows_per)],
                    out_hbm.at[c, pl.ds(stripe, rows_per)])


_BN = 2000


def _dense1_body(x_ref, w_ref, b_ref, p_ref, r_ref):
    acc = jnp.dot(x_ref[...], w_ref[...], preferred_element_type=jnp.float32)
    p_ref[...] = acc[:, :PW]
    r_ref[...] = acc[:, PW:] + b_ref[...]


def _mid_body(pp_ref, r1_ref, w_ref, b_ref, p2_ref, r2_ref):
    pa = pp_ref[0]
    pb = pp_ref[1]
    ssum = pa[:, :H] + pb[:, :H]
    cnt = pa[:, H:H + 1] + pb[:, H:H + 1]
    h = jnp.maximum(ssum / jnp.maximum(cnt, 1.0) + r1_ref[...], 0.0)
    acc = jnp.dot(h, w_ref[...], preferred_element_type=jnp.float32)
    p2_ref[...] = acc[:, :PW]
    r2_ref[...] = acc[:, PW:] + b_ref[...]


def _head_body(pp_ref, r2_ref, g_ref, wh_ref, wg_ref, bh_ref, o_ref):
    pa = pp_ref[0]
    pb = pp_ref[1]
    ssum = pa[:, :H] + pb[:, :H]
    cnt = pa[:, H:H + 1] + pb[:, H:H + 1]
    h = jnp.maximum(ssum / jnp.maximum(cnt, 1.0) + r2_ref[...], 0.0)
    o_ref[...] = (jnp.dot(h, wh_ref[...], preferred_element_type=jnp.float32)
                  + jnp.dot(g_ref[...], wg_ref[...],
                            preferred_element_type=jnp.float32)
                  + bh_ref[...])


def _row_spec(width):
    return pl.BlockSpec((_BN, width), lambda i: (i, 0))


def _full_spec(shape):
    return pl.BlockSpec(shape, lambda i: tuple(0 for _ in shape))


_GRID = (N // _BN,)

_dense1 = pl.pallas_call(
    _dense1_body,
    grid=_GRID,
    in_specs=[_row_spec(DIN), _full_spec((DIN, PW + H)), _full_spec((1, H))],
    out_specs=[_row_spec(PW), _row_spec(H)],
    out_shape=[jax.ShapeDtypeStruct((N, PW), jnp.float32),
               jax.ShapeDtypeStruct((N, H), jnp.float32)],
)

_mid = pl.pallas_call(
    _mid_body,
    grid=_GRID,
    in_specs=[pl.BlockSpec((NC, _BN, 2 * H), lambda i: (0, i, 0)),
              _row_spec(H), _full_spec((H, PW + H)), _full_spec((1, H))],
    out_specs=[_row_spec(PW), _row_spec(H)],
    out_shape=[jax.ShapeDtypeStruct((N, PW), jnp.float32),
               jax.ShapeDtypeStruct((N, H), jnp.float32)],
)

_head = pl.pallas_call(
    _head_body,
    grid=_GRID,
    in_specs=[pl.BlockSpec((NC, _BN, 2 * H), lambda i: (0, i, 0)),
              _row_spec(H), _row_spec(GF), _full_spec((H, 1)),
              _full_spec((GF, 1)), _full_spec((1, 1))],
    out_specs=_row_spec(1),
    out_shape=jax.ShapeDtypeStruct((N, 1), jnp.float32),
)


def kernel(x, edge_index, edge_attr, graph_features,
           A1, b1, Wr1, br1, A2, b2, Wr2, br2, Wh, bh):
    W1 = jnp.concatenate([
        A1.reshape(DE, DIN, H).transpose(1, 0, 2).reshape(DIN, DE * H),
        b1.reshape(DIN, H), Wr1], axis=1)
    W2 = jnp.concatenate([
        A2.reshape(DE, H, H).transpose(1, 0, 2).reshape(H, DE * H),
        b2.reshape(H, H), Wr2], axis=1)

    src = edge_index[0]
    dst = edge_index[1]
    zeros = jnp.zeros((NP, 2 * H), jnp.float32)

    p1, root1 = _dense1(x, W1, br1.reshape(1, H))
    part1 = _edge_pass(p1, src, dst, edge_attr, zeros)
    p2, root2 = _mid(part1, root1, W2, br2.reshape(1, H))
    part2 = _edge_pass(p2, src, dst, edge_attr, zeros)
    return _head(part2, root2, graph_features.T,
                 Wh[:H], Wh[H:], bh.reshape(1, 1))


# speedup vs baseline: 5.3460x; 1.0698x over previous
import functools

import jax
import jax.numpy as jnp
from jax import lax
from jax.experimental import pallas as pl
from jax.experimental.pallas import tpu as pltpu
from jax.experimental.pallas import tpu_sc as plsc

N = 10000
E = 160000
DIN = 128
H = 16
DE = 16
GF = 8

PW = DE * H + H
C = 128
NP = 10240

_info = plsc.get_sparse_core_info()
NC, NS = _info.num_cores, _info.num_subcores
NW = NC * NS


@functools.partial(
    pl.kernel,
    out_type=jax.ShapeDtypeStruct((NC, NP, 2 * H), jnp.float32),
    mesh=plsc.VectorSubcoreMesh(core_axis_name="c", subcore_axis_name="s"),
    scratch_types=[
        [pltpu.VMEM((2, C), jnp.int32)] * 2,
        [pltpu.VMEM((C, PW), jnp.float32)] * 2,
        [pltpu.VMEM((C, DE), jnp.float32)] * 2,
        [pltpu.VMEM((C, 2 * H), jnp.float32)] * 2,
        pltpu.VMEM((NP // NS, 2 * H), jnp.float32),
        pltpu.VMEM_SHARED((NP, 2 * H), jnp.float32),
        [pltpu.SemaphoreType.DMA] * 2,
    ],
    compiler_params=pltpu.CompilerParams(use_tc_tiling_on_sc=False),
)
def _edge_pass(p_hbm, ei_hbm, ea_hbm, out_hbm,
               ei_v, rows_v, ea_v, msg_v, zbuf_v, acc_sh, sem):
    c = lax.axis_index("c")
    s = lax.axis_index("s")
    wid = s * NC + c

    rows_per = NP // NS
    stripe = pl.multiple_of(s * rows_per, 8)
    zvec = jnp.zeros((H,), jnp.float32)

    def zero_body(i, carry):
        zbuf_v[i, pl.ds(0, H)] = zvec
        zbuf_v[i, pl.ds(H, H)] = zvec
        return carry

    lax.fori_loop(0, rows_per, zero_body, 0, unroll=4)
    pltpu.sync_copy(zbuf_v, acc_sh.at[pl.ds(stripe, rows_per)])

    cvec = jnp.where(lax.iota(jnp.int32, H) == 0,
                     jnp.float32(1.0), jnp.float32(0.0))

    def init_body(e, carry):
        msg_v[0][e, pl.ds(H, H)] = cvec
        msg_v[1][e, pl.ds(H, H)] = cvec
        return carry

    lax.fori_loop(0, C, init_body, 0)
    plsc.subcore_barrier()

    nchunks = E // C
    niter = (nchunks + NW - 1) // NW

    def start(j, b):
        cid = wid + j * NW

        @pl.when(cid < nchunks)
        def _():
            base = pl.multiple_of(cid * C, C)
            pltpu.sync_copy(ei_hbm.at[:, pl.ds(base, C)], ei_v[b])
            pltpu.sync_copy(ea_hbm.at[pl.ds(base, C)], ea_v[b])
            pltpu.async_copy(p_hbm.at[ei_v[b].at[0]], rows_v[b], sem[b])

    def process(j, b):
        cid = wid + j * NW

        @pl.when(cid < nchunks)
        def _():
            pltpu.make_async_copy(p_hbm.at[ei_v[b].at[0]], rows_v[b],
                                  sem[b]).wait()

            def edge_body(e, carry2):
                acc = rows_v[b][e, pl.ds(DE * H, H)]
                eav = ea_v[b][e, pl.ds(0, DE)]
                for k in range(DE):
                    acc = acc + eav[k] * rows_v[b][e, pl.ds(k * H, H)]
                msg_v[b][e, pl.ds(0, H)] = acc
                return carry2

            lax.fori_loop(0, C, edge_body, 0, unroll=2)
            pltpu.sync_copy(msg_v[b], acc_sh.at[ei_v[b].at[1]], add=True)

    start(0, 0)
    start(1, 1)

    def chunk_body(t, carry):
        j = 2 * t
        process(j, 0)
        start(j + 2, 0)
        process(j + 1, 1)
        start(j + 3, 1)
        return carry

    lax.fori_loop(0, niter // 2, chunk_body, 0)
    plsc.subcore_barrier()

    pltpu.sync_copy(acc_sh.at[pl.ds(stripe, rows_per)],
                    out_hbm.at[c, pl.ds(stripe, rows_per)])


_BN = 2000


def _dense1_body(x_ref, w_ref, b_ref, p_ref, r_ref):
    acc = jnp.dot(x_ref[...], w_ref[...], preferred_element_type=jnp.float32)
    p_ref[...] = acc[:, :PW]
    r_ref[...] = acc[:, PW:] + b_ref[...]


def _mid_body(pp_ref, r1_ref, w_ref, b_ref, p2_ref, r2_ref):
    pa = pp_ref[0]
    pb = pp_ref[1]
    ssum = pa[:, :H] + pb[:, :H]
    cnt = pa[:, H:H + 1] + pb[:, H:H + 1]
    h = jnp.maximum(ssum / jnp.maximum(cnt, 1.0) + r1_ref[...], 0.0)
    acc = jnp.dot(h, w_ref[...], preferred_element_type=jnp.float32)
    p2_ref[...] = acc[:, :PW]
    r2_ref[...] = acc[:, PW:] + b_ref[...]


def _head_body(pp_ref, r2_ref, g_ref, wh_ref, wg_ref, bh_ref, o_ref):
    pa = pp_ref[0]
    pb = pp_ref[1]
    ssum = pa[:, :H] + pb[:, :H]
    cnt = pa[:, H:H + 1] + pb[:, H:H + 1]
    h = jnp.maximum(ssum / jnp.maximum(cnt, 1.0) + r2_ref[...], 0.0)
    o_ref[...] = (jnp.dot(h, wh_ref[...], preferred_element_type=jnp.float32)
                  + jnp.dot(g_ref[...], wg_ref[...],
                            preferred_element_type=jnp.float32)
                  + bh_ref[...])


def _row_spec(width):
    return pl.BlockSpec((_BN, width), lambda i: (i, 0))


def _full_spec(shape):
    return pl.BlockSpec(shape, lambda i: tuple(0 for _ in shape))


_GRID = (N // _BN,)

_dense1 = pl.pallas_call(
    _dense1_body,
    grid=_GRID,
    in_specs=[_row_spec(DIN), _full_spec((DIN, PW + H)), _full_spec((1, H))],
    out_specs=[_row_spec(PW), _row_spec(H)],
    out_shape=[jax.ShapeDtypeStruct((N, PW), jnp.float32),
               jax.ShapeDtypeStruct((N, H), jnp.float32)],
)

_mid = pl.pallas_call(
    _mid_body,
    grid=_GRID,
    in_specs=[pl.BlockSpec((NC, _BN, 2 * H), lambda i: (0, i, 0)),
              _row_spec(H), _full_spec((H, PW + H)), _full_spec((1, H))],
    out_specs=[_row_spec(PW), _row_spec(H)],
    out_shape=[jax.ShapeDtypeStruct((N, PW), jnp.float32),
               jax.ShapeDtypeStruct((N, H), jnp.float32)],
)

_head = pl.pallas_call(
    _head_body,
    grid=_GRID,
    in_specs=[pl.BlockSpec((NC, _BN, 2 * H), lambda i: (0, i, 0)),
              _row_spec(H), _row_spec(GF), _full_spec((H, 1)),
              _full_spec((GF, 1)), _full_spec((1, 1))],
    out_specs=_row_spec(1),
    out_shape=jax.ShapeDtypeStruct((N, 1), jnp.float32),
)


def kernel(x, edge_index, edge_attr, graph_features,
           A1, b1, Wr1, br1, A2, b2, Wr2, br2, Wh, bh):
    W1 = jnp.concatenate([
        A1.reshape(DE, DIN, H).transpose(1, 0, 2).reshape(DIN, DE * H),
        b1.reshape(DIN, H), Wr1], axis=1)
    W2 = jnp.concatenate([
        A2.reshape(DE, H, H).transpose(1, 0, 2).reshape(H, DE * H),
        b2.reshape(H, H), Wr2], axis=1)

    p1, root1 = _dense1(x, W1, br1.reshape(1, H))
    part1 = _edge_pass(p1, edge_index, edge_attr)
    p2, root2 = _mid(part1, root1, W2, br2.reshape(1, H))
    part2 = _edge_pass(p2, edge_index, edge_attr)
    return _head(part2, root2, graph_features.T,
                 Wh[:H], Wh[H:], bh.reshape(1, 1))


# speedup vs baseline: 6.0695x; 1.1353x over previous
import functools

import jax
import jax.numpy as jnp
from jax import lax
from jax.experimental import pallas as pl
from jax.experimental.pallas import tpu as pltpu
from jax.experimental.pallas import tpu_sc as plsc

N = 10000
E = 160000
DIN = 128
H = 16
DE = 16
GF = 8

PW = DE * H + H
C = 128
NP = 10240

_info = plsc.get_sparse_core_info()
NC, NS = _info.num_cores, _info.num_subcores
NW = NC * NS


@functools.partial(
    pl.kernel,
    out_type=jax.ShapeDtypeStruct((NC, NP, 2 * H), jnp.float32),
    mesh=plsc.VectorSubcoreMesh(core_axis_name="c", subcore_axis_name="s"),
    scratch_types=[
        [pltpu.VMEM((2, C), jnp.int32)] * 2,
        [pltpu.VMEM((C, PW), jnp.float32)] * 2,
        [pltpu.VMEM((C, DE), jnp.float32)] * 2,
        [pltpu.VMEM((C, 2 * H), jnp.float32)] * 2,
        pltpu.VMEM((NP // NS, 2 * H), jnp.float32),
        pltpu.VMEM_SHARED((NP, 2 * H), jnp.float32),
        [pltpu.SemaphoreType.DMA] * 2,
    ],
    compiler_params=pltpu.CompilerParams(use_tc_tiling_on_sc=False),
)
def _edge_pass(p_hbm, ei_hbm, ea_hbm, out_hbm,
               ei_v, rows_v, ea_v, msg_v, zbuf_v, acc_sh, sem):
    c = lax.axis_index("c")
    s = lax.axis_index("s")
    wid = s * NC + c

    rows_per = NP // NS
    stripe = pl.multiple_of(s * rows_per, 8)
    zvec = jnp.zeros((H,), jnp.float32)

    def zero_body(i, carry):
        zbuf_v[i, pl.ds(0, H)] = zvec
        zbuf_v[i, pl.ds(H, H)] = zvec
        return carry

    lax.fori_loop(0, rows_per, zero_body, 0, unroll=4)
    pltpu.sync_copy(zbuf_v, acc_sh.at[pl.ds(stripe, rows_per)])

    cvec = jnp.where(lax.iota(jnp.int32, H) == 0,
                     jnp.float32(1.0), jnp.float32(0.0))

    def init_body(e, carry):
        msg_v[0][e, pl.ds(H, H)] = cvec
        msg_v[1][e, pl.ds(H, H)] = cvec
        return carry

    lax.fori_loop(0, C, init_body, 0)
    plsc.subcore_barrier()

    nchunks = E // C
    niter = (nchunks + NW - 1) // NW

    def start(j, b):
        cid = wid + j * NW

        @pl.when(cid < nchunks)
        def _():
            base = pl.multiple_of(cid * C, C)
            pltpu.sync_copy(ei_hbm.at[:, pl.ds(base, C)], ei_v[b])
            pltpu.sync_copy(ea_hbm.at[pl.ds(base, C)], ea_v[b])
            pltpu.async_copy(p_hbm.at[ei_v[b].at[0]], rows_v[b], sem[b])

    def process(j, b):
        cid = wid + j * NW

        @pl.when(cid < nchunks)
        def _():
            pltpu.make_async_copy(p_hbm.at[ei_v[b].at[0]], rows_v[b],
                                  sem[b]).wait()

            def edge_body(e, carry2):
                eav = ea_v[b][e, pl.ds(0, DE)]
                a0 = rows_v[b][e, pl.ds(DE * H, H)]
                a1 = eav[0] * rows_v[b][e, pl.ds(0, H)]
                a2 = eav[1] * rows_v[b][e, pl.ds(H, H)]
                a3 = eav[2] * rows_v[b][e, pl.ds(2 * H, H)]
                for k in range(3, DE - 1, 4):
                    a0 = a0 + eav[k] * rows_v[b][e, pl.ds(k * H, H)]
                    a1 = a1 + eav[k + 1] * rows_v[b][e, pl.ds((k + 1) * H, H)]
                    a2 = a2 + eav[k + 2] * rows_v[b][e, pl.ds((k + 2) * H, H)]
                    a3 = a3 + eav[k + 3] * rows_v[b][e, pl.ds((k + 3) * H, H)]
                a0 = a0 + eav[15] * rows_v[b][e, pl.ds(15 * H, H)]
                msg_v[b][e, pl.ds(0, H)] = (a0 + a1) + (a2 + a3)
                return carry2

            lax.fori_loop(0, C, edge_body, 0, unroll=4)
            pltpu.sync_copy(msg_v[b], acc_sh.at[ei_v[b].at[1]], add=True)

    start(0, 0)
    start(1, 1)

    def chunk_body(t, carry):
        j = 2 * t
        process(j, 0)
        start(j + 2, 0)
        process(j + 1, 1)
        start(j + 3, 1)
        return carry

    lax.fori_loop(0, niter // 2, chunk_body, 0)
    plsc.subcore_barrier()

    pltpu.sync_copy(acc_sh.at[pl.ds(stripe, rows_per)],
                    out_hbm.at[c, pl.ds(stripe, rows_per)])


_BN = 2000


def _dense1_body(x_ref, w_ref, b_ref, p_ref, r_ref):
    acc = jnp.dot(x_ref[...], w_ref[...], preferred_element_type=jnp.float32)
    p_ref[...] = acc[:, :PW]
    r_ref[...] = acc[:, PW:] + b_ref[...]


def _mid_body(pp_ref, r1_ref, w_ref, b_ref, p2_ref, r2_ref):
    pa = pp_ref[0]
    pb = pp_ref[1]
    ssum = pa[:, :H] + pb[:, :H]
    cnt = pa[:, H:H + 1] + pb[:, H:H + 1]
    h = jnp.maximum(ssum / jnp.maximum(cnt, 1.0) + r1_ref[...], 0.0)
    acc = jnp.dot(h, w_ref[...], preferred_element_type=jnp.float32)
    p2_ref[...] = acc[:, :PW]
    r2_ref[...] = acc[:, PW:] + b_ref[...]


def _head_body(pp_ref, r2_ref, g_ref, wh_ref, wg_ref, bh_ref, o_ref):
    pa = pp_ref[0]
    pb = pp_ref[1]
    ssum = pa[:, :H] + pb[:, :H]
    cnt = pa[:, H:H + 1] + pb[:, H:H + 1]
    h = jnp.maximum(ssum / jnp.maximum(cnt, 1.0) + r2_ref[...], 0.0)
    o_ref[...] = (jnp.dot(h, wh_ref[...], preferred_element_type=jnp.float32)
                  + jnp.dot(g_ref[...], wg_ref[...],
                            preferred_element_type=jnp.float32)
                  + bh_ref[...])


def _row_spec(width):
    return pl.BlockSpec((_BN, width), lambda i: (i, 0))


def _full_spec(shape):
    return pl.BlockSpec(shape, lambda i: tuple(0 for _ in shape))


_GRID = (N // _BN,)

_dense1 = pl.pallas_call(
    _dense1_body,
    grid=_GRID,
    in_specs=[_row_spec(DIN), _full_spec((DIN, PW + H)), _full_spec((1, H))],
    out_specs=[_row_spec(PW), _row_spec(H)],
    out_shape=[jax.ShapeDtypeStruct((N, PW), jnp.float32),
               jax.ShapeDtypeStruct((N, H), jnp.float32)],
)

_mid = pl.pallas_call(
    _mid_body,
    grid=_GRID,
    in_specs=[pl.BlockSpec((NC, _BN, 2 * H), lambda i: (0, i, 0)),
              _row_spec(H), _full_spec((H, PW + H)), _full_spec((1, H))],
    out_specs=[_row_spec(PW), _row_spec(H)],
    out_shape=[jax.ShapeDtypeStruct((N, PW), jnp.float32),
               jax.ShapeDtypeStruct((N, H), jnp.float32)],
)

_head = pl.pallas_call(
    _head_body,
    grid=_GRID,
    in_specs=[pl.BlockSpec((NC, _BN, 2 * H), lambda i: (0, i, 0)),
              _row_spec(H), _row_spec(GF), _full_spec((H, 1)),
              _full_spec((GF, 1)), _full_spec((1, 1))],
    out_specs=_row_spec(1),
    out_shape=jax.ShapeDtypeStruct((N, 1), jnp.float32),
)


def kernel(x, edge_index, edge_attr, graph_features,
           A1, b1, Wr1, br1, A2, b2, Wr2, br2, Wh, bh):
    W1 = jnp.concatenate([
        A1.reshape(DE, DIN, H).transpose(1, 0, 2).reshape(DIN, DE * H),
        b1.reshape(DIN, H), Wr1], axis=1)
    W2 = jnp.concatenate([
        A2.reshape(DE, H, H).transpose(1, 0, 2).reshape(H, DE * H),
        b2.reshape(H, H), Wr2], axis=1)

    p1, root1 = _dense1(x, W1, br1.reshape(1, H))
    part1 = _edge_pass(p1, edge_index, edge_attr)
    p2, root2 = _mid(part1, root1, W2, br2.reshape(1, H))
    part2 = _edge_pass(p2, edge_index, edge_attr)
    return _head(part2, root2, graph_features.T,
                 Wh[:H], Wh[H:], bh.reshape(1, 1))
